# baseline (speedup 1.0000x reference)
import jax
import jax.numpy as jnp
from jax import lax
from jax.experimental import pallas as pl
from jax.experimental.pallas import tpu as pltpu
from jax.experimental.pallas import tpu_sc as plsc

N_NODES = 10000
N_INC = 320000
D = 128
DH = 64
EPS = 1e-5

NC, NS, L = 2, 16, 16
N_PAD = 10240
SLAB = N_PAD // NS
CH = 256
NCHUNKS = N_INC // CH
KMAX = -(-NCHUNKS // NS)
PAIRS = (KMAX - 1) // 2
PC = 160

_mesh = plsc.VectorSubcoreMesh(core_axis_name="c", subcore_axis_name="s")


def _deg_body(edge, ones_rows, zslab16, dinv_out, binv_out,
              acc_sh, idq, ones_v, vb16, sem):
    c = lax.axis_index("c")
    s = lax.axis_index("s")
    pltpu.sync_copy(ones_rows, ones_v)
    pltpu.sync_copy(zslab16, acc_sh.at[pl.ds(s * SLAB, SLAB)])
    plsc.subcore_barrier()

    def count(row, dst):
        def chunk(k, _):
            cid = k * NS + s

            @pl.when(cid < NCHUNKS)
            def _():
                pltpu.sync_copy(edge.at[row].at[pl.ds(cid * CH, CH)], idq)
                pltpu.sync_copy(ones_v, acc_sh.at[idq], add=True)
            return 0
        lax.fori_loop(0, KMAX, chunk, 0)
        plsc.subcore_barrier()
        pltpu.sync_copy(acc_sh.at[pl.ds(s * SLAB, SLAB)], vb16)

        def inv(r, _):
            v = vb16[r, :]
            vb16[r, :] = jnp.where(v > 0.0, 1.0 / v, 0.0)
            return 0
        lax.fori_loop(0, SLAB, inv, 0)
        pltpu.sync_copy(vb16, dst.at[pl.ds(s * SLAB, SLAB)])

    @pl.when(c == 0)
    def _():
        count(0, dinv_out)

    @pl.when(c == 1)
    def _():
        count(1, binv_out)


_deg_call = pl.kernel(
    _deg_body,
    out_type=(jax.ShapeDtypeStruct((N_PAD, L), jnp.float32),
              jax.ShapeDtypeStruct((N_PAD, L), jnp.float32)),
    mesh=_mesh,
    compiler_params=pltpu.CompilerParams(use_tc_tiling_on_sc=False),
    scratch_types=[
        pltpu.VMEM_SHARED((N_PAD, L), jnp.float32),
        pltpu.VMEM((CH,), jnp.int32),
        pltpu.VMEM((CH, L), jnp.float32),
        pltpu.VMEM((SLAB, L), jnp.float32),
        pltpu.SemaphoreType.DMA,
    ],
)


def _agg_body(xt_lo, xt_hi, edge, binv_w, zslab, out_lo, out_hi,
              xt_sh, e_sh, idxa, idxb, rowsa, rowsb, vbuf, bw,
              sga, ssa, sgb, ssb):
    c = lax.axis_index("c")
    s = lax.axis_index("s")
    slab = pl.ds(s * SLAB, SLAB)
    o_sh = xt_sh

    def hop(src_tab, dst_tab, gi, si):
        pltpu.sync_copy(edge.at[:, pl.ds(s * CH, CH)], idxa)
        pltpu.async_copy(src_tab.at[idxa.at[gi]], rowsa, sga)

        def pair(k2, _):
            b = (2 * k2 + 1) * NS + s
            a2 = (2 * k2 + 2) * NS + s

            @pl.when(k2 > 0)
            def _():
                pltpu.make_async_copy(rowsb, dst_tab.at[idxb.at[si]], ssb).wait()
            pltpu.sync_copy(edge.at[:, pl.ds(b * CH, CH)], idxb)
            pltpu.make_async_copy(src_tab.at[idxa.at[gi]], rowsa, sga).wait()
            pltpu.async_copy(rowsa, dst_tab.at[idxa.at[si]], ssa, add=True)
            pltpu.async_copy(src_tab.at[idxb.at[gi]], rowsb, sgb)
            pltpu.make_async_copy(rowsa, dst_tab.at[idxa.at[si]], ssa).wait()

            @pl.when(k2 < PAIRS - 1)
            def _():
                pltpu.sync_copy(edge.at[:, pl.ds(a2 * CH, CH)], idxa)
                pltpu.async_copy(src_tab.at[idxa.at[gi]], rowsa, sga)
            pltpu.make_async_copy(src_tab.at[idxb.at[gi]], rowsb, sgb).wait()
            pltpu.async_copy(rowsb, dst_tab.at[idxb.at[si]], ssb, add=True)
            return 0
        lax.fori_loop(0, PAIRS, pair, 0)
        pltpu.make_async_copy(rowsb, dst_tab.at[idxb.at[si]], ssb).wait()

        tail = 2 * PAIRS * NS + s

        @pl.when(tail < NCHUNKS)
        def _():
            pltpu.sync_copy(edge.at[:, pl.ds(tail * CH, CH)], idxa)
            pltpu.async_copy(src_tab.at[idxa.at[gi]], rowsa, sga).wait()
            pltpu.sync_copy(rowsa, dst_tab.at[idxa.at[si]], add=True)

    def run(src, dst):
        pltpu.sync_copy(src.at[slab], xt_sh.at[slab])
        pltpu.sync_copy(zslab, e_sh.at[slab])
        plsc.subcore_barrier()

        hop(xt_sh, e_sh, 0, 1)
        plsc.subcore_barrier()

        pltpu.sync_copy(zslab, o_sh.at[slab])
        for p in range(SLAB // PC):
            seg = pl.ds(s * SLAB + p * PC, PC)
            pltpu.sync_copy(binv_w.at[seg], bw)
            pltpu.sync_copy(e_sh.at[seg], vbuf)

            def crow(r, _):
                sp = bw[r, :]
                for j in range(DH // L):
                    vbuf[r, pl.ds(j * L, L)] = vbuf[r, pl.ds(j * L, L)] * sp
                return 0
            lax.fori_loop(0, PC, crow, 0)
            pltpu.sync_copy(vbuf, e_sh.at[seg])
        plsc.subcore_barrier()

        hop(e_sh, o_sh, 1, 0)
        plsc.subcore_barrier()

        pltpu.sync_copy(o_sh.at[slab], dst.at[slab])

    @pl.when(c == 0)
    def _():
        run(xt_lo, out_lo)

    @pl.when(c == 1)
    def _():
        run(xt_hi, out_hi)


_agg_call = pl.kernel(
    _agg_body,
    out_type=(jax.ShapeDtypeStruct((N_PAD, DH), jnp.float32),
              jax.ShapeDtypeStruct((N_PAD, DH), jnp.float32)),
    mesh=_mesh,
    compiler_params=pltpu.CompilerParams(use_tc_tiling_on_sc=False),
    scratch_types=[
        pltpu.VMEM_SHARED((N_PAD, DH), jnp.float32),
        pltpu.VMEM_SHARED((N_PAD, DH), jnp.float32),
        pltpu.VMEM((2, CH), jnp.int32),
        pltpu.VMEM((2, CH), jnp.int32),
        pltpu.VMEM((CH, DH), jnp.float32),
        pltpu.VMEM((CH, DH), jnp.float32),
        pltpu.VMEM((PC, DH), jnp.float32),
        pltpu.VMEM((PC, L), jnp.float32),
        pltpu.SemaphoreType.DMA,
        pltpu.SemaphoreType.DMA,
        pltpu.SemaphoreType.DMA,
        pltpu.SemaphoreType.DMA,
    ],
)


_BR = 640
_GRID = N_PAD // _BR


def _mm_plain_body(h_ref, w_ref, lo_ref, hi_ref):
    xt = jnp.dot(h_ref[...], w_ref[...], preferred_element_type=jnp.float32)
    lo_ref[...] = xt[:, :DH]
    hi_ref[...] = xt[:, DH:]


_mm_plain = pl.pallas_call(
    _mm_plain_body,
    grid=(_GRID,),
    in_specs=[
        pl.BlockSpec((_BR, D), lambda i: (i, 0)),
        pl.BlockSpec((D, D), lambda i: (0, 0)),
    ],
    out_specs=[
        pl.BlockSpec((_BR, DH), lambda i: (i, 0)),
        pl.BlockSpec((_BR, DH), lambda i: (i, 0)),
    ],
    out_shape=(jax.ShapeDtypeStruct((N_PAD, DH), jnp.float32),
               jax.ShapeDtypeStruct((N_PAD, DH), jnp.float32)),
)


def _stats_body(lo_ref, hi_ref, dinv_ref, b_ref, h_ref, s1_ref, s2_ref):
    i = pl.program_id(0)
    a = jnp.concatenate([lo_ref[...], hi_ref[...]], axis=1)
    hb = jnp.maximum(a * dinv_ref[...] + b_ref[...], 0.0)
    h_ref[...] = hb

    @pl.when(i == 0)
    def _():
        s1_ref[...] = jnp.zeros_like(s1_ref)
        s2_ref[...] = jnp.zeros_like(s2_ref)

    rows = lax.broadcasted_iota(jnp.int32, (_BR, 1), 0) + i * _BR
    hm = jnp.where(rows < N_NODES, hb, 0.0)
    s1_ref[...] += jnp.sum(hm, axis=0, keepdims=True)
    s2_ref[...] += jnp.sum(hm * hm, axis=0, keepdims=True)


_stats = pl.pallas_call(
    _stats_body,
    grid=(_GRID,),
    in_specs=[
        pl.BlockSpec((_BR, DH), lambda i: (i, 0)),
        pl.BlockSpec((_BR, DH), lambda i: (i, 0)),
        pl.BlockSpec((_BR, 1), lambda i: (i, 0)),
        pl.BlockSpec((1, D), lambda i: (0, 0)),
    ],
    out_specs=[
        pl.BlockSpec((_BR, D), lambda i: (i, 0)),
        pl.BlockSpec((1, D), lambda i: (0, 0)),
        pl.BlockSpec((1, D), lambda i: (0, 0)),
    ],
    out_shape=(jax.ShapeDtypeStruct((N_PAD, D), jnp.float32),
               jax.ShapeDtypeStruct((1, D), jnp.float32),
               jax.ShapeDtypeStruct((1, D), jnp.float32)),
)


def _mm_bn_body(h_ref, s1_ref, s2_ref, g_ref, be_ref, w_ref, lo_ref, hi_ref):
    n = jnp.float32(N_NODES)
    mu = s1_ref[...] / n
    var = s2_ref[...] / n - mu * mu
    inv = lax.rsqrt(var + EPS)
    colA = g_ref[...] * inv
    colB = be_ref[...] - mu * colA
    hb = h_ref[...] * colA + colB
    xt = jnp.dot(hb, w_ref[...], preferred_element_type=jnp.float32)
    lo_ref[...] = xt[:, :DH]
    hi_ref[...] = xt[:, DH:]


_mm_bn = pl.pallas_call(
    _mm_bn_body,
    grid=(_GRID,),
    in_specs=[
        pl.BlockSpec((_BR, D), lambda i: (i, 0)),
        pl.BlockSpec((1, D), lambda i: (0, 0)),
        pl.BlockSpec((1, D), lambda i: (0, 0)),
        pl.BlockSpec((1, D), lambda i: (0, 0)),
        pl.BlockSpec((1, D), lambda i: (0, 0)),
        pl.BlockSpec((D, D), lambda i: (0, 0)),
    ],
    out_specs=[
        pl.BlockSpec((_BR, DH), lambda i: (i, 0)),
        pl.BlockSpec((_BR, DH), lambda i: (i, 0)),
    ],
    out_shape=(jax.ShapeDtypeStruct((N_PAD, DH), jnp.float32),
               jax.ShapeDtypeStruct((N_PAD, DH), jnp.float32)),
)


def _final_body(lo_ref, hi_ref, dinv_ref, b_ref, o_ref):
    a = jnp.concatenate([lo_ref[...], hi_ref[...]], axis=1)
    o_ref[...] = jnp.maximum(a * dinv_ref[...] + b_ref[...], 0.0)


_final = pl.pallas_call(
    _final_body,
    grid=(_GRID,),
    in_specs=[
        pl.BlockSpec((_BR, DH), lambda i: (i, 0)),
        pl.BlockSpec((_BR, DH), lambda i: (i, 0)),
        pl.BlockSpec((_BR, 1), lambda i: (i, 0)),
        pl.BlockSpec((1, D), lambda i: (0, 0)),
    ],
    out_specs=pl.BlockSpec((_BR, D), lambda i: (i, 0)),
    out_shape=jax.ShapeDtypeStruct((N_PAD, D), jnp.float32),
)


def _branch(f_pad, edge, binv_w, zslab, dinv2, Ws, bs, gammas, betas, ci, bi):
    lo, hi = _mm_plain(f_pad, Ws[ci])
    alo, ahi = _agg_call(lo, hi, edge, binv_w, zslab)
    h, s1, s2 = _stats(alo, ahi, dinv2, bs[ci][None, :])
    lo, hi = _mm_bn(h, s1, s2, gammas[bi][None, :], betas[bi][None, :], Ws[ci + 1])
    alo, ahi = _agg_call(lo, hi, edge, binv_w, zslab)
    h, s1, s2 = _stats(alo, ahi, dinv2, bs[ci + 1][None, :])
    lo, hi = _mm_bn(h, s1, s2, gammas[bi + 1][None, :], betas[bi + 1][None, :], Ws[ci + 2])
    alo, ahi = _agg_call(lo, hi, edge, binv_w, zslab)
    return _final(alo, ahi, dinv2, bs[ci + 2][None, :])[:N_NODES]


def kernel(x, y, z, edge, Ws, bs, gammas, betas):
    dinv_w, binv_w = _deg_call(edge,
                               jnp.ones((CH, L), jnp.float32),
                               jnp.zeros((SLAB, L), jnp.float32))
    dinv2 = dinv_w[:, :1]
    zslab = jnp.zeros((SLAB, DH), jnp.float32)
    pad = ((0, N_PAD - N_NODES), (0, 0))
    args = (edge, binv_w, zslab, dinv2, Ws, bs, gammas, betas)
    xo = _branch(jnp.pad(x, pad), *args, 0, 0)
    yo = _branch(jnp.pad(y, pad), *args, 3, 2)
    zo = _branch(jnp.pad(z, pad), *args, 6, 4)
    return (xo, yo, zo)


# speedup vs baseline: 1.2498x; 1.2498x over previous
import jax
import jax.numpy as jnp
from jax import lax
from jax.experimental import pallas as pl
from jax.experimental.pallas import tpu as pltpu
from jax.experimental.pallas import tpu_sc as plsc

N_NODES = 10000
N_INC = 320000
D = 128
DH = 64
EPS = 1e-5

NC, NS, L = 2, 16, 16
N_PAD = 10240
SLAB = N_PAD // NS
CH = 640
NCHUNKS = N_INC // CH
KMAX = -(-NCHUNKS // NS)
PC = 64

_mesh = plsc.VectorSubcoreMesh(core_axis_name="c", subcore_axis_name="s")


def _deg_body(edge, ones_rows, zslab16, dinv_out, binv_out,
              acc_sh, idq, ones_v, vb16, sem):
    c = lax.axis_index("c")
    s = lax.axis_index("s")
    pltpu.sync_copy(ones_rows, ones_v)
    pltpu.sync_copy(zslab16, acc_sh.at[pl.ds(s * SLAB, SLAB)])
    plsc.subcore_barrier()

    def count(row, dst):
        def chunk(k, _):
            cid = k * NS + s

            @pl.when(cid < NCHUNKS)
            def _():
                pltpu.sync_copy(edge.at[row].at[pl.ds(cid * CH, CH)], idq)
                pltpu.sync_copy(ones_v, acc_sh.at[idq], add=True)
            return 0
        lax.fori_loop(0, KMAX, chunk, 0)
        plsc.subcore_barrier()
        pltpu.sync_copy(acc_sh.at[pl.ds(s * SLAB, SLAB)], vb16)

        def inv(r, _):
            v = vb16[r, :]
            vb16[r, :] = jnp.where(v > 0.0, 1.0 / v, 0.0)
            return 0
        lax.fori_loop(0, SLAB, inv, 0)
        pltpu.sync_copy(vb16, dst.at[pl.ds(s * SLAB, SLAB)])

    @pl.when(c == 0)
    def _():
        count(0, dinv_out)

    @pl.when(c == 1)
    def _():
        count(1, binv_out)


_deg_call = pl.kernel(
    _deg_body,
    out_type=(jax.ShapeDtypeStruct((N_PAD, L), jnp.float32),
              jax.ShapeDtypeStruct((N_PAD, L), jnp.float32)),
    mesh=_mesh,
    compiler_params=pltpu.CompilerParams(use_tc_tiling_on_sc=False),
    scratch_types=[
        pltpu.VMEM_SHARED((N_PAD, L), jnp.float32),
        pltpu.VMEM((CH,), jnp.int32),
        pltpu.VMEM((CH, L), jnp.float32),
        pltpu.VMEM((SLAB, L), jnp.float32),
        pltpu.SemaphoreType.DMA,
    ],
)


def _agg_body(xt_lo, xt_hi, edge, binv_w, zslab, out_lo, out_hi,
              xt_sh, e_sh, idx2, rows, vbuf, bw, sem):
    c = lax.axis_index("c")
    s = lax.axis_index("s")
    slab = pl.ds(s * SLAB, SLAB)
    o_sh = xt_sh

    def hop(src_tab, dst_tab, gi, si):
        def chunk(k, _):
            cid = k * NS + s

            @pl.when(cid < NCHUNKS)
            def _():
                pltpu.sync_copy(edge.at[:, pl.ds(cid * CH, CH)], idx2)
                pltpu.async_copy(src_tab.at[idx2.at[gi]], rows, sem).wait()
                pltpu.sync_copy(rows, dst_tab.at[idx2.at[si]], add=True)
            return 0
        lax.fori_loop(0, KMAX, chunk, 0)

    def run(src, dst):
        pltpu.sync_copy(src.at[slab], xt_sh.at[slab])
        pltpu.sync_copy(zslab, e_sh.at[slab])
        plsc.subcore_barrier()

        hop(xt_sh, e_sh, 0, 1)
        plsc.subcore_barrier()

        pltpu.sync_copy(zslab, o_sh.at[slab])
        for p in range(SLAB // PC):
            seg = pl.ds(s * SLAB + p * PC, PC)
            pltpu.sync_copy(binv_w.at[seg], bw)
            pltpu.sync_copy(e_sh.at[seg], vbuf)

            def crow(r, _):
                sp = bw[r, :]
                for j in range(DH // L):
                    vbuf[r, pl.ds(j * L, L)] = vbuf[r, pl.ds(j * L, L)] * sp
                return 0
            lax.fori_loop(0, PC, crow, 0)
            pltpu.sync_copy(vbuf, e_sh.at[seg])
        plsc.subcore_barrier()

        hop(e_sh, o_sh, 1, 0)
        plsc.subcore_barrier()

        pltpu.sync_copy(o_sh.at[slab], dst.at[slab])

    @pl.when(c == 0)
    def _():
        run(xt_lo, out_lo)

    @pl.when(c == 1)
    def _():
        run(xt_hi, out_hi)


_agg_call = pl.kernel(
    _agg_body,
    out_type=(jax.ShapeDtypeStruct((N_PAD, DH), jnp.float32),
              jax.ShapeDtypeStruct((N_PAD, DH), jnp.float32)),
    mesh=_mesh,
    compiler_params=pltpu.CompilerParams(use_tc_tiling_on_sc=False),
    scratch_types=[
        pltpu.VMEM_SHARED((N_PAD, DH), jnp.float32),
        pltpu.VMEM_SHARED((N_PAD, DH), jnp.float32),
        pltpu.VMEM((2, CH), jnp.int32),
        pltpu.VMEM((CH, DH), jnp.float32),
        pltpu.VMEM((PC, DH), jnp.float32),
        pltpu.VMEM((PC, L), jnp.float32),
        pltpu.SemaphoreType.DMA,
    ],
)


_BR = 640
_GRID = N_PAD // _BR


def _mm_plain_body(h_ref, w_ref, lo_ref, hi_ref):
    xt = jnp.dot(h_ref[...], w_ref[...], preferred_element_type=jnp.float32)
    lo_ref[...] = xt[:, :DH]
    hi_ref[...] = xt[:, DH:]


_mm_plain = pl.pallas_call(
    _mm_plain_body,
    grid=(_GRID,),
    in_specs=[
        pl.BlockSpec((_BR, D), lambda i: (i, 0)),
        pl.BlockSpec((D, D), lambda i: (0, 0)),
    ],
    out_specs=[
        pl.BlockSpec((_BR, DH), lambda i: (i, 0)),
        pl.BlockSpec((_BR, DH), lambda i: (i, 0)),
    ],
    out_shape=(jax.ShapeDtypeStruct((N_PAD, DH), jnp.float32),
               jax.ShapeDtypeStruct((N_PAD, DH), jnp.float32)),
)


def _stats_body(lo_ref, hi_ref, dinv_ref, b_ref, h_ref, s1_ref, s2_ref):
    i = pl.program_id(0)
    a = jnp.concatenate([lo_ref[...], hi_ref[...]], axis=1)
    hb = jnp.maximum(a * dinv_ref[...] + b_ref[...], 0.0)
    h_ref[...] = hb

    @pl.when(i == 0)
    def _():
        s1_ref[...] = jnp.zeros_like(s1_ref)
        s2_ref[...] = jnp.zeros_like(s2_ref)

    rows = lax.broadcasted_iota(jnp.int32, (_BR, 1), 0) + i * _BR
    hm = jnp.where(rows < N_NODES, hb, 0.0)
    s1_ref[...] += jnp.sum(hm, axis=0, keepdims=True)
    s2_ref[...] += jnp.sum(hm * hm, axis=0, keepdims=True)


_stats = pl.pallas_call(
    _stats_body,
    grid=(_GRID,),
    in_specs=[
        pl.BlockSpec((_BR, DH), lambda i: (i, 0)),
        pl.BlockSpec((_BR, DH), lambda i: (i, 0)),
        pl.BlockSpec((_BR, 1), lambda i: (i, 0)),
        pl.BlockSpec((1, D), lambda i: (0, 0)),
    ],
    out_specs=[
        pl.BlockSpec((_BR, D), lambda i: (i, 0)),
        pl.BlockSpec((1, D), lambda i: (0, 0)),
        pl.BlockSpec((1, D), lambda i: (0, 0)),
    ],
    out_shape=(jax.ShapeDtypeStruct((N_PAD, D), jnp.float32),
               jax.ShapeDtypeStruct((1, D), jnp.float32),
               jax.ShapeDtypeStruct((1, D), jnp.float32)),
)


def _mm_bn_body(h_ref, s1_ref, s2_ref, g_ref, be_ref, w_ref, lo_ref, hi_ref):
    n = jnp.float32(N_NODES)
    mu = s1_ref[...] / n
    var = s2_ref[...] / n - mu * mu
    inv = lax.rsqrt(var + EPS)
    colA = g_ref[...] * inv
    colB = be_ref[...] - mu * colA
    hb = h_ref[...] * colA + colB
    xt = jnp.dot(hb, w_ref[...], preferred_element_type=jnp.float32)
    lo_ref[...] = xt[:, :DH]
    hi_ref[...] = xt[:, DH:]


_mm_bn = pl.pallas_call(
    _mm_bn_body,
    grid=(_GRID,),
    in_specs=[
        pl.BlockSpec((_BR, D), lambda i: (i, 0)),
        pl.BlockSpec((1, D), lambda i: (0, 0)),
        pl.BlockSpec((1, D), lambda i: (0, 0)),
        pl.BlockSpec((1, D), lambda i: (0, 0)),
        pl.BlockSpec((1, D), lambda i: (0, 0)),
        pl.BlockSpec((D, D), lambda i: (0, 0)),
    ],
    out_specs=[
        pl.BlockSpec((_BR, DH), lambda i: (i, 0)),
        pl.BlockSpec((_BR, DH), lambda i: (i, 0)),
    ],
    out_shape=(jax.ShapeDtypeStruct((N_PAD, DH), jnp.float32),
               jax.ShapeDtypeStruct((N_PAD, DH), jnp.float32)),
)


def _final_body(lo_ref, hi_ref, dinv_ref, b_ref, o_ref):
    a = jnp.concatenate([lo_ref[...], hi_ref[...]], axis=1)
    o_ref[...] = jnp.maximum(a * dinv_ref[...] + b_ref[...], 0.0)


_final = pl.pallas_call(
    _final_body,
    grid=(_GRID,),
    in_specs=[
        pl.BlockSpec((_BR, DH), lambda i: (i, 0)),
        pl.BlockSpec((_BR, DH), lambda i: (i, 0)),
        pl.BlockSpec((_BR, 1), lambda i: (i, 0)),
        pl.BlockSpec((1, D), lambda i: (0, 0)),
    ],
    out_specs=pl.BlockSpec((_BR, D), lambda i: (i, 0)),
    out_shape=jax.ShapeDtypeStruct((N_PAD, D), jnp.float32),
)


def _branch(f_pad, edge, binv_w, zslab, dinv2, Ws, bs, gammas, betas, ci, bi):
    lo, hi = _mm_plain(f_pad, Ws[ci])
    alo, ahi = _agg_call(lo, hi, edge, binv_w, zslab)
    h, s1, s2 = _stats(alo, ahi, dinv2, bs[ci][None, :])
    lo, hi = _mm_bn(h, s1, s2, gammas[bi][None, :], betas[bi][None, :], Ws[ci + 1])
    alo, ahi = _agg_call(lo, hi, edge, binv_w, zslab)
    h, s1, s2 = _stats(alo, ahi, dinv2, bs[ci + 1][None, :])
    lo, hi = _mm_bn(h, s1, s2, gammas[bi + 1][None, :], betas[bi + 1][None, :], Ws[ci + 2])
    alo, ahi = _agg_call(lo, hi, edge, binv_w, zslab)
    return _final(alo, ahi, dinv2, bs[ci + 2][None, :])[:N_NODES]


def kernel(x, y, z, edge, Ws, bs, gammas, betas):
    dinv_w, binv_w = _deg_call(edge,
                               jnp.ones((CH, L), jnp.float32),
                               jnp.zeros((SLAB, L), jnp.float32))
    dinv2 = dinv_w[:, :1]
    zslab = jnp.zeros((SLAB, DH), jnp.float32)
    pad = ((0, N_PAD - N_NODES), (0, 0))
    args = (edge, binv_w, zslab, dinv2, Ws, bs, gammas, betas)
    xo = _branch(jnp.pad(x, pad), *args, 0, 0)
    yo = _branch(jnp.pad(y, pad), *args, 3, 2)
    zo = _branch(jnp.pad(z, pad), *args, 6, 4)
    return (xo, yo, zo)


# speedup vs baseline: 1.2744x; 1.0197x over previous
import jax
import jax.numpy as jnp
from jax import lax
from jax.experimental import pallas as pl
from jax.experimental.pallas import tpu as pltpu
from jax.experimental.pallas import tpu_sc as plsc

N_NODES = 10000
N_INC = 320000
D = 128
DH = 64
EPS = 1e-5

NC, NS, L = 2, 16, 16
N_PAD = 10240
SLAB = N_PAD // NS
CH = 640
NCHUNKS = N_INC // CH
KMAX = -(-NCHUNKS // NS)
PC = 320

_mesh = plsc.VectorSubcoreMesh(core_axis_name="c", subcore_axis_name="s")


def _deg_body(edge, ones_rows, zslab16, dinv_out, binv_out,
              acc_sh, idq, ones_v, vb16, sem):
    c = lax.axis_index("c")
    s = lax.axis_index("s")
    pltpu.sync_copy(ones_rows, ones_v)
    pltpu.sync_copy(zslab16, acc_sh.at[pl.ds(s * SLAB, SLAB)])
    plsc.subcore_barrier()

    def count(row, dst):
        def chunk(k, _):
            cid = k * NS + s

            @pl.when(cid < NCHUNKS)
            def _():
                pltpu.sync_copy(edge.at[row].at[pl.ds(cid * CH, CH)], idq)
                pltpu.sync_copy(ones_v, acc_sh.at[idq], add=True)
            return 0
        lax.fori_loop(0, KMAX, chunk, 0)
        plsc.subcore_barrier()
        pltpu.sync_copy(acc_sh.at[pl.ds(s * SLAB, SLAB)], vb16)

        def inv(r, _):
            v = vb16[r, :]
            vb16[r, :] = jnp.where(v > 0.0, 1.0 / v, 0.0)
            return 0
        lax.fori_loop(0, SLAB, inv, 0)
        pltpu.sync_copy(vb16, dst.at[pl.ds(s * SLAB, SLAB)])

    @pl.when(c == 0)
    def _():
        count(0, dinv_out)

    @pl.when(c == 1)
    def _():
        count(1, binv_out)


_deg_call = pl.kernel(
    _deg_body,
    out_type=(jax.ShapeDtypeStruct((N_PAD, L), jnp.float32),
              jax.ShapeDtypeStruct((N_PAD, L), jnp.float32)),
    mesh=_mesh,
    compiler_params=pltpu.CompilerParams(use_tc_tiling_on_sc=False),
    scratch_types=[
        pltpu.VMEM_SHARED((N_PAD, L), jnp.float32),
        pltpu.VMEM((CH,), jnp.int32),
        pltpu.VMEM((CH, L), jnp.float32),
        pltpu.VMEM((SLAB, L), jnp.float32),
        pltpu.SemaphoreType.DMA,
    ],
)


def _agg_body(xt_lo, xt_hi, edge, binv_w, zslab, out_lo, out_hi,
              xt_sh, e_sh, idx2, rows, bw, sem):
    c = lax.axis_index("c")
    s = lax.axis_index("s")
    slab = pl.ds(s * SLAB, SLAB)
    o_sh = xt_sh

    def hop(src_tab, dst_tab, gi, si):
        def chunk(k, _):
            cid = k * NS + s

            @pl.when(cid < NCHUNKS)
            def _():
                pltpu.sync_copy(edge.at[:, pl.ds(cid * CH, CH)], idx2)
                pltpu.async_copy(src_tab.at[idx2.at[gi]], rows, sem).wait()
                pltpu.sync_copy(rows, dst_tab.at[idx2.at[si]], add=True)
            return 0
        lax.fori_loop(0, KMAX, chunk, 0)

    def run(src, dst):
        pltpu.sync_copy(src.at[slab], xt_sh.at[slab])
        pltpu.sync_copy(zslab, e_sh.at[slab])
        plsc.subcore_barrier()

        hop(xt_sh, e_sh, 0, 1)
        plsc.subcore_barrier()

        pltpu.sync_copy(zslab, o_sh.at[slab])
        for p in range(SLAB // PC):
            seg = pl.ds(s * SLAB + p * PC, PC)
            pltpu.sync_copy(binv_w.at[seg], bw)
            pltpu.sync_copy(e_sh.at[seg], rows.at[pl.ds(0, PC)])

            def crow(r, _):
                sp = bw[r, :]
                for j in range(DH // L):
                    rows[r, pl.ds(j * L, L)] = rows[r, pl.ds(j * L, L)] * sp
                return 0
            lax.fori_loop(0, PC, crow, 0)
            pltpu.sync_copy(rows.at[pl.ds(0, PC)], e_sh.at[seg])
        plsc.subcore_barrier()

        hop(e_sh, o_sh, 1, 0)
        plsc.subcore_barrier()

        pltpu.sync_copy(o_sh.at[slab], dst.at[slab])

    @pl.when(c == 0)
    def _():
        run(xt_lo, out_lo)

    @pl.when(c == 1)
    def _():
        run(xt_hi, out_hi)


_agg_call = pl.kernel(
    _agg_body,
    out_type=(jax.ShapeDtypeStruct((N_PAD, DH), jnp.float32),
              jax.ShapeDtypeStruct((N_PAD, DH), jnp.float32)),
    mesh=_mesh,
    compiler_params=pltpu.CompilerParams(use_tc_tiling_on_sc=False),
    scratch_types=[
        pltpu.VMEM_SHARED((N_PAD, DH), jnp.float32),
        pltpu.VMEM_SHARED((N_PAD, DH), jnp.float32),
        pltpu.VMEM((2, CH), jnp.int32),
        pltpu.VMEM((CH, DH), jnp.float32),
        pltpu.VMEM((PC, L), jnp.float32),
        pltpu.SemaphoreType.DMA,
    ],
)


_BR = 640
_GRID = N_PAD // _BR


def _mm_plain_body(h_ref, w_ref, lo_ref, hi_ref):
    xt = jnp.dot(h_ref[...], w_ref[...], preferred_element_type=jnp.float32)
    lo_ref[...] = xt[:, :DH]
    hi_ref[...] = xt[:, DH:]


_mm_plain = pl.pallas_call(
    _mm_plain_body,
    grid=(_GRID,),
    in_specs=[
        pl.BlockSpec((_BR, D), lambda i: (i, 0)),
        pl.BlockSpec((D, D), lambda i: (0, 0)),
    ],
    out_specs=[
        pl.BlockSpec((_BR, DH), lambda i: (i, 0)),
        pl.BlockSpec((_BR, DH), lambda i: (i, 0)),
    ],
    out_shape=(jax.ShapeDtypeStruct((N_PAD, DH), jnp.float32),
               jax.ShapeDtypeStruct((N_PAD, DH), jnp.float32)),
)


def _stats_body(lo_ref, hi_ref, dinv_ref, b_ref, h_ref, s1_ref, s2_ref):
    i = pl.program_id(0)
    a = jnp.concatenate([lo_ref[...], hi_ref[...]], axis=1)
    hb = jnp.maximum(a * dinv_ref[...] + b_ref[...], 0.0)
    h_ref[...] = hb

    @pl.when(i == 0)
    def _():
        s1_ref[...] = jnp.zeros_like(s1_ref)
        s2_ref[...] = jnp.zeros_like(s2_ref)

    rows = lax.broadcasted_iota(jnp.int32, (_BR, 1), 0) + i * _BR
    hm = jnp.where(rows < N_NODES, hb, 0.0)
    s1_ref[...] += jnp.sum(hm, axis=0, keepdims=True)
    s2_ref[...] += jnp.sum(hm * hm, axis=0, keepdims=True)


_stats = pl.pallas_call(
    _stats_body,
    grid=(_GRID,),
    in_specs=[
        pl.BlockSpec((_BR, DH), lambda i: (i, 0)),
        pl.BlockSpec((_BR, DH), lambda i: (i, 0)),
        pl.BlockSpec((_BR, 1), lambda i: (i, 0)),
        pl.BlockSpec((1, D), lambda i: (0, 0)),
    ],
    out_specs=[
        pl.BlockSpec((_BR, D), lambda i: (i, 0)),
        pl.BlockSpec((1, D), lambda i: (0, 0)),
        pl.BlockSpec((1, D), lambda i: (0, 0)),
    ],
    out_shape=(jax.ShapeDtypeStruct((N_PAD, D), jnp.float32),
               jax.ShapeDtypeStruct((1, D), jnp.float32),
               jax.ShapeDtypeStruct((1, D), jnp.float32)),
)


def _mm_bn_body(h_ref, s1_ref, s2_ref, g_ref, be_ref, w_ref, lo_ref, hi_ref):
    n = jnp.float32(N_NODES)
    mu = s1_ref[...] / n
    var = s2_ref[...] / n - mu * mu
    inv = lax.rsqrt(var + EPS)
    colA = g_ref[...] * inv
    colB = be_ref[...] - mu * colA
    hb = h_ref[...] * colA + colB
    xt = jnp.dot(hb, w_ref[...], preferred_element_type=jnp.float32)
    lo_ref[...] = xt[:, :DH]
    hi_ref[...] = xt[:, DH:]


_mm_bn = pl.pallas_call(
    _mm_bn_body,
    grid=(_GRID,),
    in_specs=[
        pl.BlockSpec((_BR, D), lambda i: (i, 0)),
        pl.BlockSpec((1, D), lambda i: (0, 0)),
        pl.BlockSpec((1, D), lambda i: (0, 0)),
        pl.BlockSpec((1, D), lambda i: (0, 0)),
        pl.BlockSpec((1, D), lambda i: (0, 0)),
        pl.BlockSpec((D, D), lambda i: (0, 0)),
    ],
    out_specs=[
        pl.BlockSpec((_BR, DH), lambda i: (i, 0)),
        pl.BlockSpec((_BR, DH), lambda i: (i, 0)),
    ],
    out_shape=(jax.ShapeDtypeStruct((N_PAD, DH), jnp.float32),
               jax.ShapeDtypeStruct((N_PAD, DH), jnp.float32)),
)


def _final_body(lo_ref, hi_ref, dinv_ref, b_ref, o_ref):
    a = jnp.concatenate([lo_ref[...], hi_ref[...]], axis=1)
    o_ref[...] = jnp.maximum(a * dinv_ref[...] + b_ref[...], 0.0)


_final = pl.pallas_call(
    _final_body,
    grid=(_GRID,),
    in_specs=[
        pl.BlockSpec((_BR, DH), lambda i: (i, 0)),
        pl.BlockSpec((_BR, DH), lambda i: (i, 0)),
        pl.BlockSpec((_BR, 1), lambda i: (i, 0)),
        pl.BlockSpec((1, D), lambda i: (0, 0)),
    ],
    out_specs=pl.BlockSpec((_BR, D), lambda i: (i, 0)),
    out_shape=jax.ShapeDtypeStruct((N_PAD, D), jnp.float32),
)


def _branch(f_pad, edge, binv_w, zslab, dinv2, Ws, bs, gammas, betas, ci, bi):
    lo, hi = _mm_plain(f_pad, Ws[ci])
    alo, ahi = _agg_call(lo, hi, edge, binv_w, zslab)
    h, s1, s2 = _stats(alo, ahi, dinv2, bs[ci][None, :])
    lo, hi = _mm_bn(h, s1, s2, gammas[bi][None, :], betas[bi][None, :], Ws[ci + 1])
    alo, ahi = _agg_call(lo, hi, edge, binv_w, zslab)
    h, s1, s2 = _stats(alo, ahi, dinv2, bs[ci + 1][None, :])
    lo, hi = _mm_bn(h, s1, s2, gammas[bi + 1][None, :], betas[bi + 1][None, :], Ws[ci + 2])
    alo, ahi = _agg_call(lo, hi, edge, binv_w, zslab)
    return _final(alo, ahi, dinv2, bs[ci + 2][None, :])[:N_NODES]


def kernel(x, y, z, edge, Ws, bs, gammas, betas):
    dinv_w, binv_w = _deg_call(edge,
                               jnp.ones((CH, L), jnp.float32),
                               jnp.zeros((SLAB, L), jnp.float32))
    dinv2 = dinv_w[:, :1]
    zslab = jnp.zeros((SLAB, DH), jnp.float32)
    pad = ((0, N_PAD - N_NODES), (0, 0))
    args = (edge, binv_w, zslab, dinv2, Ws, bs, gammas, betas)
    xo = _branch(jnp.pad(x, pad), *args, 0, 0)
    yo = _branch(jnp.pad(y, pad), *args, 3, 2)
    zo = _branch(jnp.pad(z, pad), *args, 6, 4)
    return (xo, yo, zo)


# speedup vs baseline: 1.2781x; 1.0029x over previous
import jax
import jax.numpy as jnp
from jax import lax
from jax.experimental import pallas as pl
from jax.experimental.pallas import tpu as pltpu
from jax.experimental.pallas import tpu_sc as plsc

N_NODES = 10000
N_INC = 320000
D = 128
DH = 64
EPS = 1e-5

NC, NS, L = 2, 16, 16
N_PAD = 10240
SLAB = N_PAD // NS
CH = 640
NCHUNKS = N_INC // CH
KMAX = -(-NCHUNKS // NS)
PC = 320

_mesh = plsc.VectorSubcoreMesh(core_axis_name="c", subcore_axis_name="s")


def _deg_body(edge, ones_rows, zslab16, dinv_out, binv_out,
              acc_sh, idq, ones_v, vb16, sem):
    c = lax.axis_index("c")
    s = lax.axis_index("s")
    pltpu.sync_copy(ones_rows, ones_v)
    pltpu.sync_copy(zslab16, acc_sh.at[pl.ds(s * SLAB, SLAB)])
    plsc.subcore_barrier()

    def count(row, dst):
        def chunk(k, _):
            cid = k * NS + s

            @pl.when(cid < NCHUNKS)
            def _():
                pltpu.sync_copy(edge.at[row].at[pl.ds(cid * CH, CH)], idq)
                pltpu.sync_copy(ones_v, acc_sh.at[idq], add=True)
            return 0
        lax.fori_loop(0, KMAX, chunk, 0)
        plsc.subcore_barrier()
        pltpu.sync_copy(acc_sh.at[pl.ds(s * SLAB, SLAB)], vb16)

        def inv(r, _):
            v = vb16[r, :]
            vb16[r, :] = jnp.where(v > 0.0, 1.0 / v, 0.0)
            return 0
        lax.fori_loop(0, SLAB, inv, 0)
        pltpu.sync_copy(vb16, dst.at[pl.ds(s * SLAB, SLAB)])

    @pl.when(c == 0)
    def _():
        count(0, dinv_out)

    @pl.when(c == 1)
    def _():
        count(1, binv_out)


_deg_call = pl.kernel(
    _deg_body,
    out_type=(jax.ShapeDtypeStruct((N_PAD, L), jnp.float32),
              jax.ShapeDtypeStruct((N_PAD, L), jnp.float32)),
    mesh=_mesh,
    compiler_params=pltpu.CompilerParams(use_tc_tiling_on_sc=False),
    scratch_types=[
        pltpu.VMEM_SHARED((N_PAD, L), jnp.float32),
        pltpu.VMEM((CH,), jnp.int32),
        pltpu.VMEM((CH, L), jnp.float32),
        pltpu.VMEM((SLAB, L), jnp.float32),
        pltpu.SemaphoreType.DMA,
    ],
)


def _agg_body(xt_lo, xt_hi, edge, binv_w, zslab, out_lo, out_hi,
              xt_sh, e_sh, idx2, rows, bw, sem):
    c = lax.axis_index("c")
    s = lax.axis_index("s")
    slab = pl.ds(s * SLAB, SLAB)
    o_sh = xt_sh

    def hop(src_tab, dst_tab, gi, si):
        def chunk(k, _):
            cid = k * NS + s

            @pl.when(cid < NCHUNKS)
            def _():
                pltpu.sync_copy(edge.at[:, pl.ds(cid * CH, CH)], idx2)
                pltpu.async_copy(src_tab.at[idx2.at[gi]], rows, sem).wait()
                pltpu.sync_copy(rows, dst_tab.at[idx2.at[si]], add=True)
            return 0
        lax.fori_loop(0, KMAX, chunk, 0)

    def run(src, dst):
        pltpu.sync_copy(src.at[slab], xt_sh.at[slab])
        pltpu.sync_copy(zslab, e_sh.at[slab])
        plsc.subcore_barrier()

        hop(xt_sh, e_sh, 0, 1)
        plsc.subcore_barrier()

        pltpu.sync_copy(zslab, o_sh.at[slab])
        for p in range(SLAB // PC):
            seg = pl.ds(s * SLAB + p * PC, PC)
            pltpu.sync_copy(binv_w.at[seg], bw)
            pltpu.sync_copy(e_sh.at[seg], rows.at[pl.ds(0, PC)])

            def crow(r, _):
                sp = bw[r, :]
                for j in range(DH // L):
                    rows[r, pl.ds(j * L, L)] = rows[r, pl.ds(j * L, L)] * sp
                return 0
            lax.fori_loop(0, PC, crow, 0)
            pltpu.sync_copy(rows.at[pl.ds(0, PC)], e_sh.at[seg])
        plsc.subcore_barrier()

        hop(e_sh, o_sh, 1, 0)
        plsc.subcore_barrier()

        pltpu.sync_copy(o_sh.at[slab], dst.at[slab])

    @pl.when(c == 0)
    def _():
        run(xt_lo, out_lo)

    @pl.when(c == 1)
    def _():
        run(xt_hi, out_hi)


_agg_call = pl.kernel(
    _agg_body,
    out_type=(jax.ShapeDtypeStruct((N_PAD, DH), jnp.float32),
              jax.ShapeDtypeStruct((N_PAD, DH), jnp.float32)),
    mesh=_mesh,
    compiler_params=pltpu.CompilerParams(use_tc_tiling_on_sc=False),
    scratch_types=[
        pltpu.VMEM_SHARED((N_PAD, DH), jnp.float32),
        pltpu.VMEM_SHARED((N_PAD, DH), jnp.float32),
        pltpu.VMEM((2, CH), jnp.int32),
        pltpu.VMEM((CH, DH), jnp.float32),
        pltpu.VMEM((PC, L), jnp.float32),
        pltpu.SemaphoreType.DMA,
    ],
)


_BR = 640
_GRID = N_PAD // _BR


def _mm_plain_body(h_ref, w_ref, lo_ref, hi_ref):
    xt = jnp.dot(h_ref[...], w_ref[...], preferred_element_type=jnp.float32)
    lo_ref[...] = xt[:, :DH]
    hi_ref[...] = xt[:, DH:]


_mm_plain = pl.pallas_call(
    _mm_plain_body,
    grid=(_GRID,),
    in_specs=[
        pl.BlockSpec((_BR, D), lambda i: (i, 0)),
        pl.BlockSpec((D, D), lambda i: (0, 0)),
    ],
    out_specs=[
        pl.BlockSpec((_BR, DH), lambda i: (i, 0)),
        pl.BlockSpec((_BR, DH), lambda i: (i, 0)),
    ],
    out_shape=(jax.ShapeDtypeStruct((N_PAD, DH), jnp.float32),
               jax.ShapeDtypeStruct((N_PAD, DH), jnp.float32)),
)


def _bnmm_body(lo_ref, hi_ref, dinv_ref, b_ref, g_ref, be_ref, w_ref,
               olo_ref, ohi_ref, h_ref, s1_ref, s2_ref):
    p = pl.program_id(0)
    i = pl.program_id(1)

    @pl.when(p == 0)
    def _():
        a = jnp.concatenate([lo_ref[...], hi_ref[...]], axis=1)
        hb = jnp.maximum(a * dinv_ref[...] + b_ref[...], 0.0)
        h_ref[pl.ds(i * _BR, _BR), :] = hb

        @pl.when(i == 0)
        def _():
            s1_ref[...] = jnp.zeros_like(s1_ref)
            s2_ref[...] = jnp.zeros_like(s2_ref)

        rows = lax.broadcasted_iota(jnp.int32, (_BR, 1), 0) + i * _BR
        hm = jnp.where(rows < N_NODES, hb, 0.0)
        s1_ref[...] += jnp.sum(hm, axis=0, keepdims=True)
        s2_ref[...] += jnp.sum(hm * hm, axis=0, keepdims=True)

    @pl.when(p == 1)
    def _():
        n = jnp.float32(N_NODES)
        mu = s1_ref[...] / n
        var = s2_ref[...] / n - mu * mu
        inv = lax.rsqrt(var + EPS)
        colA = g_ref[...] * inv
        colB = be_ref[...] - mu * colA
        hb = h_ref[pl.ds(i * _BR, _BR), :] * colA + colB
        xt = jnp.dot(hb, w_ref[...], preferred_element_type=jnp.float32)
        olo_ref[...] = xt[:, :DH]
        ohi_ref[...] = xt[:, DH:]


_bnmm = pl.pallas_call(
    _bnmm_body,
    grid=(2, _GRID),
    in_specs=[
        pl.BlockSpec((_BR, DH), lambda p, i: (i, 0)),
        pl.BlockSpec((_BR, DH), lambda p, i: (i, 0)),
        pl.BlockSpec((_BR, 1), lambda p, i: (i, 0)),
        pl.BlockSpec((1, D), lambda p, i: (0, 0)),
        pl.BlockSpec((1, D), lambda p, i: (0, 0)),
        pl.BlockSpec((1, D), lambda p, i: (0, 0)),
        pl.BlockSpec((D, D), lambda p, i: (0, 0)),
    ],
    out_specs=[
        pl.BlockSpec((_BR, DH), lambda p, i: (i, 0)),
        pl.BlockSpec((_BR, DH), lambda p, i: (i, 0)),
    ],
    out_shape=(jax.ShapeDtypeStruct((N_PAD, DH), jnp.float32),
               jax.ShapeDtypeStruct((N_PAD, DH), jnp.float32)),
    scratch_shapes=[
        pltpu.VMEM((N_PAD, D), jnp.float32),
        pltpu.VMEM((1, D), jnp.float32),
        pltpu.VMEM((1, D), jnp.float32),
    ],
)


def _final_body(lo_ref, hi_ref, dinv_ref, b_ref, o_ref):
    a = jnp.concatenate([lo_ref[...], hi_ref[...]], axis=1)
    o_ref[...] = jnp.maximum(a * dinv_ref[...] + b_ref[...], 0.0)


_final = pl.pallas_call(
    _final_body,
    grid=(_GRID,),
    in_specs=[
        pl.BlockSpec((_BR, DH), lambda i: (i, 0)),
        pl.BlockSpec((_BR, DH), lambda i: (i, 0)),
        pl.BlockSpec((_BR, 1), lambda i: (i, 0)),
        pl.BlockSpec((1, D), lambda i: (0, 0)),
    ],
    out_specs=pl.BlockSpec((_BR, D), lambda i: (i, 0)),
    out_shape=jax.ShapeDtypeStruct((N_PAD, D), jnp.float32),
)


def _branch(f_pad, edge, binv_w, zslab, dinv2, Ws, bs, gammas, betas, ci, bi):
    lo, hi = _mm_plain(f_pad, Ws[ci])
    alo, ahi = _agg_call(lo, hi, edge, binv_w, zslab)
    lo, hi = _bnmm(alo, ahi, dinv2, bs[ci][None, :],
                   gammas[bi][None, :], betas[bi][None, :], Ws[ci + 1])
    alo, ahi = _agg_call(lo, hi, edge, binv_w, zslab)
    lo, hi = _bnmm(alo, ahi, dinv2, bs[ci + 1][None, :],
                   gammas[bi + 1][None, :], betas[bi + 1][None, :], Ws[ci + 2])
    alo, ahi = _agg_call(lo, hi, edge, binv_w, zslab)
    return _final(alo, ahi, dinv2, bs[ci + 2][None, :])[:N_NODES]


def kernel(x, y, z, edge, Ws, bs, gammas, betas):
    dinv_w, binv_w = _deg_call(edge,
                               jnp.ones((CH, L), jnp.float32),
                               jnp.zeros((SLAB, L), jnp.float32))
    dinv2 = dinv_w[:, :1]
    zslab = jnp.zeros((SLAB, DH), jnp.float32)
    pad = ((0, N_PAD - N_NODES), (0, 0))
    args = (edge, binv_w, zslab, dinv2, Ws, bs, gammas, betas)
    xo = _branch(jnp.pad(x, pad), *args, 0, 0)
    yo = _branch(jnp.pad(y, pad), *args, 3, 2)
    zo = _branch(jnp.pad(z, pad), *args, 6, 4)
    return (xo, yo, zo)
